# Initial kernel scaffold; baseline (speedup 1.0000x reference)
import jax
import jax.numpy as jnp
from jax.experimental import pallas as pl


def kernel(node, edges, edges_attr, W0, b0, W1, b1, W2, b2, W3, b3, Wfc, bfc):
    raise NotImplementedError("write your pallas kernel here")


# speedup vs baseline: 11.6183x; 11.6183x over previous
import functools

import jax
import jax.numpy as jnp
from jax import lax
from jax.experimental import pallas as pl
from jax.experimental.pallas import tpu as pltpu
from jax.experimental.pallas import tpu_sc as plsc

N = 10000
D = 128
N_PAD = 10240
DUMP = N
NW = 32
C = 128
E_ALL = 320000 + N
CHUNKS = -(-E_ALL // (NW * C))
EPT = CHUNKS * C
E_PAD = EPT * NW
ROWS_PT = N_PAD // 16
BLK = 512
NBLK = N_PAD // BLK

@functools.cache
def _mesh():
    return plsc.VectorSubcoreMesh(core_axis_name="c", subcore_axis_name="s",
                                  num_cores=2, num_subcores=16)


def _zero_rows(rows_v):
    def z(i, carry):
        for j in range(D // 16):
            rows_v[i, pl.ds(j * 16, 16)] = jnp.zeros((16,), jnp.float32)
        return carry
    lax.fori_loop(0, C, z, 0)


def _copy_row16(dst_v, src_v, i):
    for j in range(C // 16):
        dst_v[pl.ds(j * 16, 16)] = src_v[i, pl.ds(j * 16, 16)]


@functools.cache
def _build_deg_kernel():
    return functools.partial(
        pl.kernel,
        out_type=jax.ShapeDtypeStruct((2, N_PAD, D), jnp.float32),
        mesh=_mesh(),
        scratch_types=[
            pltpu.VMEM((CHUNKS, C), jnp.int32),
            pltpu.VMEM((C,), jnp.int32),
            pltpu.VMEM((C, D), jnp.float32),
            pltpu.VMEM_SHARED((N_PAD, D), jnp.float32),
        ],
    )(_deg_body)


def _deg_body(dst_hbm, out_hbm, idx_all, idx_v, buf_v, acc_sh):
    c = lax.axis_index("c")
    s = lax.axis_index("s")
    w = s * 2 + c

    pltpu.sync_copy(dst_hbm.at[w], idx_all)

    def fill(val):
        def z(i, carry):
            for j in range(D // 16):
                buf_v[i, pl.ds(j * 16, 16)] = jnp.full((16,), val, jnp.float32)
            return carry
        lax.fori_loop(0, C, z, 0)

    fill(0.0)
    base = s * ROWS_PT
    for r in range(ROWS_PT // C):
        pltpu.sync_copy(buf_v, acc_sh.at[pl.ds(base + r * C, C)])
    plsc.subcore_barrier()
    fill(1.0)

    def body(i, carry):
        _copy_row16(idx_v, idx_all, i)
        pltpu.sync_copy(buf_v, acc_sh.at[idx_v], add=True)
        return carry
    lax.fori_loop(0, CHUNKS, body, 0)
    plsc.subcore_barrier()

    for r in range(ROWS_PT // C):
        pltpu.sync_copy(acc_sh.at[pl.ds(base + r * C, C)], buf_v)
        pltpu.sync_copy(buf_v, out_hbm.at[c, pl.ds(base + r * C, C)])


@functools.cache
def _build_msg_kernel():
    return functools.partial(
        pl.kernel,
        out_type=jax.ShapeDtypeStruct((2, N_PAD, D), jnp.float32),
        mesh=_mesh(),
        scratch_types=[
            pltpu.VMEM((CHUNKS, C), jnp.int32),
            pltpu.VMEM((CHUNKS, C), jnp.int32),
            pltpu.VMEM((C,), jnp.int32),
            pltpu.VMEM((C,), jnp.int32),
            pltpu.VMEM((C, D), jnp.float32),
            pltpu.VMEM_SHARED((N_PAD, D), jnp.float32),
            pltpu.SemaphoreType.DMA,
        ],
    )(_msg_body)


def _msg_body(y_hbm, src_hbm, dst_hbm, out_hbm,
              sidx_all, didx_all, sidx_v, didx_v, rows_v, acc_sh, sem):
    c = lax.axis_index("c")
    s = lax.axis_index("s")
    w = s * 2 + c

    pltpu.sync_copy(src_hbm.at[w], sidx_all)
    pltpu.sync_copy(dst_hbm.at[w], didx_all)

    _zero_rows(rows_v)
    base = s * ROWS_PT
    for r in range(ROWS_PT // C):
        pltpu.sync_copy(rows_v, acc_sh.at[pl.ds(base + r * C, C)])
    plsc.subcore_barrier()

    def body(i, carry):
        _copy_row16(sidx_v, sidx_all, i)
        _copy_row16(didx_v, didx_all, i)
        pltpu.async_copy(y_hbm.at[sidx_v], rows_v, sem).wait()
        pltpu.sync_copy(rows_v, acc_sh.at[didx_v], add=True)
        return carry
    lax.fori_loop(0, CHUNKS, body, 0)
    plsc.subcore_barrier()

    for r in range(ROWS_PT // C):
        pltpu.sync_copy(acc_sh.at[pl.ds(base + r * C, C)], rows_v)
        pltpu.sync_copy(rows_v, out_hbm.at[c, pl.ds(base + r * C, C)])


def _pre_body(degp_ref, dinv_ref):
    x = degp_ref[...]
    d = x[0, :, 0:1] + x[1, :, 0:1]
    dinv_ref[...] = jnp.where(d > 0, lax.rsqrt(d), 0.0)


def _dinv(deg_partials):
    return pl.pallas_call(
        _pre_body,
        grid=(NBLK,),
        in_specs=[pl.BlockSpec((2, BLK, D), lambda i: (0, i, 0))],
        out_specs=pl.BlockSpec((BLK, 1), lambda i: (i, 0)),
        out_shape=jax.ShapeDtypeStruct((N_PAD, 1), jnp.float32),
    )(deg_partials)


def _first_body(x_ref, w_ref, d_ref, y_ref):
    xw = jnp.dot(x_ref[...], w_ref[...], preferred_element_type=jnp.float32)
    y_ref[...] = xw * d_ref[...]


def _first(x, W, dinv):
    return pl.pallas_call(
        _first_body,
        grid=(NBLK,),
        in_specs=[
            pl.BlockSpec((BLK, D), lambda i: (i, 0)),
            pl.BlockSpec((D, D), lambda i: (0, 0)),
            pl.BlockSpec((BLK, 1), lambda i: (i, 0)),
        ],
        out_specs=pl.BlockSpec((BLK, D), lambda i: (i, 0)),
        out_shape=jax.ShapeDtypeStruct((N_PAD, D), jnp.float32),
    )(x, W, dinv)


def _make_layer(has_res):
    def body(p_ref, d_ref, b_ref, *rest):
        if has_res:
            xprev_ref, w_ref, xnew_ref, y_ref = rest
        else:
            w_ref, xnew_ref, y_ref = rest
        d = d_ref[...]
        agg = (p_ref[0] + p_ref[1]) * d + b_ref[...]
        if has_res:
            agg = agg + xprev_ref[...]
        xn = jnp.maximum(agg, 0.0)
        xnew_ref[...] = xn
        y_ref[...] = jnp.dot(xn, w_ref[...],
                             preferred_element_type=jnp.float32) * d
    return body


def _layer(p, dinv, b, xprev, W):
    has_res = xprev is not None
    in_specs = [
        pl.BlockSpec((2, BLK, D), lambda i: (0, i, 0)),
        pl.BlockSpec((BLK, 1), lambda i: (i, 0)),
        pl.BlockSpec((1, D), lambda i: (0, 0)),
    ]
    args = [p, dinv, b]
    if has_res:
        in_specs.append(pl.BlockSpec((BLK, D), lambda i: (i, 0)))
        args.append(xprev)
    in_specs.append(pl.BlockSpec((D, D), lambda i: (0, 0)))
    args.append(W)
    return pl.pallas_call(
        _make_layer(has_res),
        grid=(NBLK,),
        in_specs=in_specs,
        out_specs=[
            pl.BlockSpec((BLK, D), lambda i: (i, 0)),
            pl.BlockSpec((BLK, D), lambda i: (i, 0)),
        ],
        out_shape=[
            jax.ShapeDtypeStruct((N_PAD, D), jnp.float32),
            jax.ShapeDtypeStruct((N_PAD, D), jnp.float32),
        ],
    )(*args)


def _final_body(p_ref, d_ref, b_ref, xprev_ref, wfc_ref, bfc_ref,
                out_ref, acc_ref):
    i = pl.program_id(0)
    d = d_ref[...]
    xn = jnp.maximum((p_ref[0] + p_ref[1]) * d + b_ref[...] + xprev_ref[...],
                     0.0)
    rows = i * BLK + lax.broadcasted_iota(jnp.int32, (BLK, D), 0)
    xn = jnp.where(rows < N, xn, 0.0)
    part = jnp.sum(xn, axis=0, keepdims=True)

    @pl.when(i == 0)
    def _():
        acc_ref[...] = jnp.zeros_like(acc_ref)

    acc_ref[0:1, :] += part

    @pl.when(i == NBLK - 1)
    def _():
        tot = acc_ref[0:1, :] * (1.0 / N)
        out_ref[...] = jnp.dot(tot, wfc_ref[...],
                               preferred_element_type=jnp.float32) + bfc_ref[...]


def _final(p, dinv, b, xprev, wfc_pad, bfc_pad):
    return pl.pallas_call(
        _final_body,
        grid=(NBLK,),
        in_specs=[
            pl.BlockSpec((2, BLK, D), lambda i: (0, i, 0)),
            pl.BlockSpec((BLK, 1), lambda i: (i, 0)),
            pl.BlockSpec((1, D), lambda i: (0, 0)),
            pl.BlockSpec((BLK, D), lambda i: (i, 0)),
            pl.BlockSpec((D, D), lambda i: (0, 0)),
            pl.BlockSpec((1, D), lambda i: (0, 0)),
        ],
        out_specs=pl.BlockSpec((1, D), lambda i: (0, 0)),
        out_shape=jax.ShapeDtypeStruct((1, D), jnp.float32),
        scratch_shapes=[pltpu.VMEM((8, D), jnp.float32)],
    )(p, dinv, b, xprev, wfc_pad, bfc_pad)


def _msg(y, src, dst):
    return _build_msg_kernel()(y, src, dst)


def kernel(node, edges, edges_attr, W0, b0, W1, b1, W2, b2, W3, b3, Wfc, bfc):
    del edges_attr
    loop = jnp.arange(N, dtype=edges.dtype)
    pad = jnp.full((E_PAD - E_ALL,), DUMP, dtype=edges.dtype)
    src = jnp.concatenate([edges[0], loop, pad]).reshape(NW, CHUNKS, C)
    dst = jnp.concatenate([edges[1], loop, pad]).reshape(NW, CHUNKS, C)

    node_p = jnp.pad(node, ((0, N_PAD - N), (0, 0)))
    wfc_pad = jnp.pad(Wfc, ((0, 0), (0, D - Wfc.shape[1])))
    bfc_pad = jnp.pad(bfc, (0, D - bfc.shape[0])).reshape(1, D)
    b0r = b0.reshape(1, D)
    b1r = b1.reshape(1, D)
    b2r = b2.reshape(1, D)
    b3r = b3.reshape(1, D)

    deg_p = _build_deg_kernel()(dst)
    dinv = _dinv(deg_p)

    y0 = _first(node_p, W0, dinv)
    p = _msg(y0, src, dst)
    x1, y1 = _layer(p, dinv, b0r, None, W1)
    p = _msg(y1, src, dst)
    x2, y2 = _layer(p, dinv, b1r, x1, W2)
    p = _msg(y2, src, dst)
    x3, y3 = _layer(p, dinv, b2r, x2, W3)
    p = _msg(y3, src, dst)
    out = _final(p, dinv, b3r, x3, wfc_pad, bfc_pad)
    return out[:, :2]
